# Initial kernel scaffold; baseline (speedup 1.0000x reference)
import jax
import jax.numpy as jnp
from jax.experimental import pallas as pl


def _index_points2(points, idx):
    B = points.shape[0]
    return points[jnp.arange(B)[:, None], idx]


def _index_points3(points, idx):
    B = points.shape[0]
    return points[jnp.arange(B)[:, None, None], idx]


def _fps_elementwise(x, y, z, npoint):
    B, N = x.shape
    bidx = jnp.arange(B)
    iota = jnp.arange(N, dtype=jnp.int32)[None, :]

    def step(state, _):
        distance, farthest = state
        cx = x[bidx, farthest][:, None]
        cy = y[bidx, farthest][:, None]
        cz = z[bidx, farthest][:, None]
        dx = x - cx
        dy = y - cy
        dz = z - cz
        dist = (dx * dx + dy * dy) + dz * dz
        distance = jnp.minimum(distance, dist)
        m = jnp.max(distance, axis=-1, keepdims=True)
        new_farthest = jnp.min(
            jnp.where(distance == m, iota, N), axis=-1
        ).astype(jnp.int32)
        return (distance, new_farthest), farthest

    init = (jnp.full((B, N), 1e10, jnp.float32), jnp.zeros((B,), jnp.int32))
    _, cents = jax.lax.scan(step, init, None, length=npoint)
    return jnp.transpose(cents)


def _query_ball_elementwise(radius, nsample, x, y, z, new_xyz):
    B, N = x.shape
    S = new_xyz.shape[1]
    sx = new_xyz[:, :, 0][:, :, None]
    sy = new_xyz[:, :, 1][:, :, None]
    sz = new_xyz[:, :, 2][:, :, None]
    dxm = x[:, None, :]
    dym = y[:, None, :]
    dzm = z[:, None, :]
    mm = (sx * dxm + sy * dym) + sz * dzm
    ns = (sx * sx + sy * sy) + sz * sz
    nd = (dxm * dxm + dym * dym) + dzm * dzm
    sqr = -2.0 * mm + ns + nd
    group_idx = jnp.broadcast_to(jnp.arange(N, dtype=jnp.int32), (B, S, N))
    group_idx = jnp.where(sqr > radius ** 2, N, group_idx)
    group_idx = jnp.sort(group_idx, axis=-1)[:, :, :nsample]
    group_first = group_idx[:, :, :1]
    group_idx = jnp.where(group_idx == N, jnp.broadcast_to(group_first, group_idx.shape), group_idx)
    return group_idx


def _bn(x, g, b):
    mean = jnp.mean(x, axis=(0, 2, 3), keepdims=True)
    var = jnp.var(x, axis=(0, 2, 3), keepdims=True)
    return (x - mean) / jnp.sqrt(var + 1e-5) * g.reshape(1, -1, 1, 1) + b.reshape(1, -1, 1, 1)


def _conv1x1(x, w, b):
    return jnp.einsum('bchw,oc->bohw', x, w[:, :, 0, 0]) + b.reshape(1, -1, 1, 1)


def _copy_kernel(x_ref, o_ref):
    o_ref[...] = x_ref[...]


def kernel(xyz, points, npoint, radius, nsample, w1, b1, bn1_g, bn1_b, nt1_w, nt1_b, nt1_g, nt1_b2, nt2_w, nt2_b, nt2_g, nt2_b2, out_w, out_b, out_g, out_b2):
    x = xyz[:, 0, :]
    y = xyz[:, 1, :]
    z = xyz[:, 2, :]
    xyz_t = jnp.transpose(xyz, (0, 2, 1))
    points_t = jnp.transpose(points, (0, 2, 1))
    fps_idx = _fps_elementwise(x, y, z, 512)
    new_xyz = _index_points2(xyz_t, fps_idx)
    idx = _query_ball_elementwise(radius, 32, x, y, z, new_xyz)
    grouped_xyz = _index_points3(xyz_t, idx)
    grouped_points = _index_points3(points_t, idx)
    gx = grouped_xyz.at[:, :, 0, :].set(0.0)
    density = jnp.sum(gx, axis=-1, keepdims=True)
    density = jnp.where(density < 1e-10, 1e-10, density)
    inv = 1.0 / density
    inv_max = jnp.max(inv, axis=2, keepdims=True)
    density_scale = inv / inv_max
    gxp = jnp.transpose(gx, (0, 3, 1, 2))
    weight = jax.nn.relu(_bn(_conv1x1(gxp, w1, b1), bn1_g, bn1_b))
    ds = jnp.transpose(density_scale, (0, 3, 1, 2))
    ds1 = jax.nn.relu(_bn(_conv1x1(ds, nt1_w, nt1_b), nt1_g, nt1_b2))
    ds = jax.nn.sigmoid(_bn(_conv1x1(ds1, nt2_w, nt2_b), nt2_g, nt2_b2))
    gf = jnp.transpose(grouped_points, (0, 3, 2, 1))
    gf = jnp.transpose(gf, (0, 1, 3, 2))
    npts = gf * ds
    npts = jnp.transpose(npts, (0, 2, 1, 3))
    wgt = jnp.transpose(weight, (0, 2, 3, 1))
    npts = jnp.matmul(npts, wgt)
    npts = jnp.transpose(npts, (0, 2, 1, 3))
    out = jnp.einsum('bcsk,ock->bos', npts, out_w[:, :, 0, :]) + out_b.reshape(1, -1, 1)
    out = out[:, :, :, None]
    out = _bn(out, out_g, out_b2)
    out = jnp.squeeze(out, axis=-1)
    out = pl.pallas_call(
        _copy_kernel,
        out_shape=jax.ShapeDtypeStruct(out.shape, out.dtype),
    )(out)
    return out


# speedup vs baseline: 1.4730x; 1.4730x over previous
import functools

import jax
import jax.numpy as jnp
from jax import lax
from jax.experimental import pallas as pl
from jax.experimental.pallas import tpu as pltpu
from jax.experimental.pallas import tpu_sc as plsc

B, N, S, K, D = 16, 4096, 512, 32, 64


def _fps_kernel(xyz_ref, cx_ref, cy_ref, cz_ref):
    x = xyz_ref[:, 0, :]
    y = xyz_ref[:, 1, :]
    z = xyz_ref[:, 2, :]
    iota = lax.broadcasted_iota(jnp.int32, (B, N), 1)
    def _t(col):
        return col.reshape(1, B)

    def step(i, carry):
        distance, farthest = carry
        sel = iota == farthest
        cx = jnp.sum(jnp.where(sel, x, 0.0), axis=1, keepdims=True)
        cy = jnp.sum(jnp.where(sel, y, 0.0), axis=1, keepdims=True)
        cz = jnp.sum(jnp.where(sel, z, 0.0), axis=1, keepdims=True)
        cx_ref[pl.ds(i, 1), :] = _t(cx)
        cy_ref[pl.ds(i, 1), :] = _t(cy)
        cz_ref[pl.ds(i, 1), :] = _t(cz)
        dx = x - cx
        dy = y - cy
        dz = z - cz
        sq1 = dx * dx
        sq2 = dy * dy
        sq3 = dz * dz
        dist = (sq1 + sq2) + sq3
        distance = jnp.minimum(distance, dist)
        m = jnp.max(distance, axis=1, keepdims=True)
        farthest = jnp.min(jnp.where(distance == m, iota, N), axis=1, keepdims=True)
        return distance, farthest

    lax.fori_loop(0, S, step,
                  (jnp.full((B, N), 1e10, jnp.float32),
                   jnp.zeros((B, 1), jnp.int32)))


def _run_fps(xyz):
    return pl.pallas_call(
        _fps_kernel,
        out_shape=[jax.ShapeDtypeStruct((S, B), jnp.float32)] * 3,
    )(xyz)


def _sqr_kernel(nx_ref, xt_ref, ns_ref, nd_ref, o_ref):
    nx = nx_ref[0]
    xt = xt_ref[0]
    mm = lax.dot_general(nx, xt, (((1,), (1,)), ((), ())),
                         preferred_element_type=jnp.float32)
    o_ref[0] = (-2.0 * mm + ns_ref[0, 0][:, None]) + nd_ref[0, 0][None, :]


def _run_sqr(new_xyz, xyz_t, ns_host, nd_host):
    return pl.pallas_call(
        _sqr_kernel,
        grid=(B,),
        in_specs=[
            pl.BlockSpec((1, S, 3), lambda b: (b, 0, 0)),
            pl.BlockSpec((1, N, 3), lambda b: (b, 0, 0)),
            pl.BlockSpec((1, 1, S), lambda b: (b, 0, 0)),
            pl.BlockSpec((1, 1, N), lambda b: (b, 0, 0)),
        ],
        out_specs=pl.BlockSpec((1, S, N), lambda b: (b, 0, 0)),
        out_shape=jax.ShapeDtypeStruct((B, S, N), jnp.float32),
    )(new_xyz, xyz_t, ns_host.reshape(B, 1, S), nd_host.reshape(B, 1, N))


def _index_points3(points, idx):
    return points[jnp.arange(B)[:, None, None], idx]


def _bn(x, g, b):
    mean = jnp.mean(x, axis=(0, 2, 3), keepdims=True)
    var = jnp.var(x, axis=(0, 2, 3), keepdims=True)
    return (x - mean) / jnp.sqrt(var + 1e-5) * g.reshape(1, -1, 1, 1) + b.reshape(1, -1, 1, 1)


def _conv1x1(x, w, b):
    return jnp.einsum('bchw,oc->bohw', x, w[:, :, 0, 0]) + b.reshape(1, -1, 1, 1)


def kernel(xyz, points, npoint, radius, nsample, w1, b1, bn1_g, bn1_b, nt1_w, nt1_b, nt1_g, nt1_b2, nt2_w, nt2_b, nt2_g, nt2_b2, out_w, out_b, out_g, out_b2):
    xyz_t = jnp.transpose(xyz, (0, 2, 1))
    points_t = jnp.transpose(points, (0, 2, 1))

    cx, cy, cz = _run_fps(xyz)
    new_xyz = jnp.stack([cx.T, cy.T, cz.T], axis=-1)

    ns_host = jnp.sum(new_xyz ** 2, -1)
    nd_host = jnp.sum(xyz_t ** 2, -1)
    sqr = _run_sqr(new_xyz, xyz_t, ns_host, nd_host)

    group_idx = jnp.broadcast_to(jnp.arange(N, dtype=jnp.int32), (B, S, N))
    group_idx = jnp.where(sqr > radius ** 2, N, group_idx)
    group_idx = jnp.sort(group_idx, axis=-1)[:, :, :K]
    group_first = group_idx[:, :, :1]
    idx = jnp.where(group_idx == N, jnp.broadcast_to(group_first, group_idx.shape), group_idx)

    grouped_xyz = _index_points3(xyz_t, idx)
    grouped_points = _index_points3(points_t, idx)
    gx = grouped_xyz.at[:, :, 0, :].set(0.0)
    density = jnp.sum(gx, axis=-1, keepdims=True)
    density = jnp.where(density < 1e-10, 1e-10, density)
    inv = 1.0 / density
    inv_max = jnp.max(inv, axis=2, keepdims=True)
    density_scale = inv / inv_max
    gxp = jnp.transpose(gx, (0, 3, 1, 2))
    weight = jax.nn.relu(_bn(_conv1x1(gxp, w1, b1), bn1_g, bn1_b))
    ds = jnp.transpose(density_scale, (0, 3, 1, 2))
    ds1 = jax.nn.relu(_bn(_conv1x1(ds, nt1_w, nt1_b), nt1_g, nt1_b2))
    ds = jax.nn.sigmoid(_bn(_conv1x1(ds1, nt2_w, nt2_b), nt2_g, nt2_b2))
    gf = jnp.transpose(grouped_points, (0, 3, 2, 1))
    gf = jnp.transpose(gf, (0, 1, 3, 2))
    npts = gf * ds
    npts = jnp.transpose(npts, (0, 2, 1, 3))
    wgt = jnp.transpose(weight, (0, 2, 3, 1))
    npts = jnp.matmul(npts, wgt)
    npts = jnp.transpose(npts, (0, 2, 1, 3))
    out = jnp.einsum('bcsk,ock->bos', npts, out_w[:, :, 0, :]) + out_b.reshape(1, -1, 1)
    out = out[:, :, :, None]
    out = _bn(out, out_g, out_b2)
    out = jnp.squeeze(out, axis=-1)
    return out


# speedup vs baseline: 2.5088x; 1.7032x over previous
import functools

import jax
import jax.numpy as jnp
from jax import lax
from jax.experimental import pallas as pl
from jax.experimental.pallas import tpu as pltpu
from jax.experimental.pallas import tpu_sc as plsc

B, N, S, K, D = 16, 4096, 512, 32, 64


def _fps_kernel(xyz_ref, cx_ref, cy_ref, cz_ref):
    x = xyz_ref[:, 0, :]
    y = xyz_ref[:, 1, :]
    z = xyz_ref[:, 2, :]
    iota = lax.broadcasted_iota(jnp.int32, (B, N), 1)
    def _t(col):
        return col.reshape(1, B)

    def step(i, carry):
        distance, farthest = carry
        sel = iota == farthest
        cx = jnp.sum(jnp.where(sel, x, 0.0), axis=1, keepdims=True)
        cy = jnp.sum(jnp.where(sel, y, 0.0), axis=1, keepdims=True)
        cz = jnp.sum(jnp.where(sel, z, 0.0), axis=1, keepdims=True)
        cx_ref[pl.ds(i, 1), :] = _t(cx)
        cy_ref[pl.ds(i, 1), :] = _t(cy)
        cz_ref[pl.ds(i, 1), :] = _t(cz)
        dx = x - cx
        dy = y - cy
        dz = z - cz
        sq1 = dx * dx
        sq2 = dy * dy
        sq3 = dz * dz
        dist = (sq1 + sq2) + sq3
        distance = jnp.minimum(distance, dist)
        m = jnp.max(distance, axis=1, keepdims=True)
        farthest = jnp.min(jnp.where(distance == m, iota, N), axis=1, keepdims=True)
        return distance, farthest

    lax.fori_loop(0, S, step,
                  (jnp.full((B, N), 1e10, jnp.float32),
                   jnp.zeros((B, 1), jnp.int32)))


def _run_fps(xyz):
    return pl.pallas_call(
        _fps_kernel,
        out_shape=[jax.ShapeDtypeStruct((S, B), jnp.float32)] * 3,
    )(xyz)


def _sqr_kernel(nx_ref, xt_ref, ns_ref, nd_ref, o_ref):
    nx = nx_ref[0]
    xt = xt_ref[0]
    mm = lax.dot_general(nx, xt, (((1,), (1,)), ((), ())),
                         preferred_element_type=jnp.float32)
    o_ref[0] = (-2.0 * mm + ns_ref[0, 0][:, None]) + nd_ref[0, 0][None, :]


def _run_sqr(new_xyz, xyz_t, ns_host, nd_host):
    return pl.pallas_call(
        _sqr_kernel,
        grid=(B,),
        in_specs=[
            pl.BlockSpec((1, S, 3), lambda b: (b, 0, 0)),
            pl.BlockSpec((1, N, 3), lambda b: (b, 0, 0)),
            pl.BlockSpec((1, 1, S), lambda b: (b, 0, 0)),
            pl.BlockSpec((1, 1, N), lambda b: (b, 0, 0)),
        ],
        out_specs=pl.BlockSpec((1, S, N), lambda b: (b, 0, 0)),
        out_shape=jax.ShapeDtypeStruct((B, S, N), jnp.float32),
    )(new_xyz, xyz_t, ns_host.reshape(B, 1, S), nd_host.reshape(B, 1, N))


NW = 32
RPW = (B * S) // NW
BLK = 1


DUMP = RPW * K


def _select_body(sqr_hbm, out_hbm, rows_v, outb_v, tmp_v):
    c = lax.axis_index("c")
    s = lax.axis_index("s")
    wid = s * 2 + c
    base_row = wid * RPW

    def row_block(rb, carry):
        row0 = base_row + rb * BLK
        pltpu.sync_copy(sqr_hbm.at[pl.ds(row0 * N, BLK * N)], rows_v)
        b_off = (row0 // S) * N

        for r8 in range(BLK):
            r_loc = rb * BLK + r8
            obase = r_loc * K

            def chunk(ci, cc_vec, _r8=r8, _ob=obase, _bo=b_off):
                iota16 = lax.iota(jnp.int32, 16)
                one = jnp.full((16,), 1.0, jnp.float32)
                fifteen = jnp.full((16,), 15, jnp.int32)
                for u in range(8):
                    off = ci * 128 + u * 16
                    d = rows_v[pl.ds(_r8 * N + off, 16)]
                    indf = jnp.minimum(jnp.maximum((one - d) * 1e30 + one, 0.0), 1.0)
                    mi = indf.astype(jnp.int32)
                    cs = plsc.cumsum(mi)
                    slot = cc_vec + cs - 1
                    sf = (jnp.full((16,), K, jnp.float32) - 0.5) - slot.astype(jnp.float32)
                    okf = jnp.minimum(jnp.maximum(sf * 1e30, 0.0), 1.0)
                    take = mi * okf.astype(jnp.int32)
                    slot_abs = ((jnp.full((16,), _ob, jnp.int32) + slot) * take
                                + (1 - take) * (DUMP + iota16))
                    val = jnp.full((16,), off + _bo, jnp.int32) + iota16
                    plsc.store_scatter(outb_v, [slot_abs], val)
                    tmp_v[pl.ds(0, 16)] = cs
                    cc_vec = cc_vec + tmp_v[pl.ds(0, 16)]
                return cc_vec

            cnt_vec = lax.fori_loop(0, N // 128, chunk, jnp.zeros((16,), jnp.int32))
            tot_splat = jnp.minimum(cnt_vec, K)
            first = outb_v[pl.ds(0, 16)]
            iota16b = lax.iota(jnp.int32, 16)
            for h in range(K // 16):
                pos = iota16b + (h * 16)
                wf = jnp.minimum(jnp.maximum(
                    (tot_splat - pos).astype(jnp.float32) * 1e30, 0.0), 1.0)
                wi = wf.astype(jnp.int32)
                cur = outb_v[pl.ds(obase + h * 16, 16)]
                outb_v[pl.ds(obase + h * 16, 16)] = cur * wi + first * (1 - wi)
        return carry

    lax.fori_loop(0, RPW // BLK, row_block, 0)
    pltpu.sync_copy(outb_v.at[pl.ds(0, RPW * K)], out_hbm.at[pl.ds(base_row * K, RPW * K)])


def _run_select(sqr_flat):
    return pl.kernel(
        _select_body,
        out_type=jax.ShapeDtypeStruct((B * S * K,), jnp.int32),
        mesh=plsc.VectorSubcoreMesh(core_axis_name="c", subcore_axis_name="s"),
        scratch_types=[
            pltpu.VMEM((BLK * N,), jnp.float32),
            pltpu.VMEM((RPW * K + 16,), jnp.int32),
            pltpu.VMEM((16,), jnp.int32),
        ],
    )(sqr_flat)


def _index_points3(points, idx):
    return points[jnp.arange(B)[:, None, None], idx]


def _bn(x, g, b):
    mean = jnp.mean(x, axis=(0, 2, 3), keepdims=True)
    var = jnp.var(x, axis=(0, 2, 3), keepdims=True)
    return (x - mean) / jnp.sqrt(var + 1e-5) * g.reshape(1, -1, 1, 1) + b.reshape(1, -1, 1, 1)


def _conv1x1(x, w, b):
    return jnp.einsum('bchw,oc->bohw', x, w[:, :, 0, 0]) + b.reshape(1, -1, 1, 1)


def kernel(xyz, points, npoint, radius, nsample, w1, b1, bn1_g, bn1_b, nt1_w, nt1_b, nt1_g, nt1_b2, nt2_w, nt2_b, nt2_g, nt2_b2, out_w, out_b, out_g, out_b2):
    xyz_t = jnp.transpose(xyz, (0, 2, 1))
    points_t = jnp.transpose(points, (0, 2, 1))

    cx, cy, cz = _run_fps(xyz)
    new_xyz = jnp.stack([cx.T, cy.T, cz.T], axis=-1)

    ns_host = jnp.sum(new_xyz ** 2, -1)
    nd_host = jnp.sum(xyz_t ** 2, -1)
    sqr = _run_sqr(new_xyz, xyz_t, ns_host, nd_host)

    key = jnp.where(sqr > radius ** 2, jnp.float32(-N),
                    -jnp.broadcast_to(jnp.arange(N, dtype=jnp.float32), (B, S, N)))
    vals, _ = lax.top_k(key, K)
    gi = (-vals).astype(jnp.int32)
    first = gi[:, :, :1]
    idx = jnp.where(gi == N, jnp.broadcast_to(first, gi.shape), gi)
    idxg = (idx + (jnp.arange(B, dtype=jnp.int32) * N)[:, None, None]).reshape(B * S, K)

    grouped_xyz = xyz_t.reshape(B * N, 3)[idxg].reshape(B, S, K, 3)
    grouped_points = points_t.reshape(B * N, D)[idxg].reshape(B, S, K, D)
    gx = grouped_xyz.at[:, :, 0, :].set(0.0)
    density = jnp.sum(gx, axis=-1, keepdims=True)
    density = jnp.where(density < 1e-10, 1e-10, density)
    inv = 1.0 / density
    inv_max = jnp.max(inv, axis=2, keepdims=True)
    density_scale = inv / inv_max
    gxp = jnp.transpose(gx, (0, 3, 1, 2))
    weight = jax.nn.relu(_bn(_conv1x1(gxp, w1, b1), bn1_g, bn1_b))
    ds = jnp.transpose(density_scale, (0, 3, 1, 2))
    ds1 = jax.nn.relu(_bn(_conv1x1(ds, nt1_w, nt1_b), nt1_g, nt1_b2))
    ds = jax.nn.sigmoid(_bn(_conv1x1(ds1, nt2_w, nt2_b), nt2_g, nt2_b2))
    gf = jnp.transpose(grouped_points, (0, 3, 2, 1))
    gf = jnp.transpose(gf, (0, 1, 3, 2))
    npts = gf * ds
    npts = jnp.transpose(npts, (0, 2, 1, 3))
    wgt = jnp.transpose(weight, (0, 2, 3, 1))
    npts = jnp.matmul(npts, wgt)
    npts = jnp.transpose(npts, (0, 2, 1, 3))
    out = jnp.einsum('bcsk,ock->bos', npts, out_w[:, :, 0, :]) + out_b.reshape(1, -1, 1)
    out = out[:, :, :, None]
    out = _bn(out, out_g, out_b2)
    out = jnp.squeeze(out, axis=-1)
    return out


# speedup vs baseline: 2.9608x; 1.1802x over previous
import functools

import jax
import jax.numpy as jnp
from jax import lax
from jax.experimental import pallas as pl
from jax.experimental.pallas import tpu as pltpu
from jax.experimental.pallas import tpu_sc as plsc

B, N, S, K, D = 16, 4096, 512, 32, 64


def _fps_kernel(xyz_ref, cx_ref, cy_ref, cz_ref):
    x = xyz_ref[:, 0, :]
    y = xyz_ref[:, 1, :]
    z = xyz_ref[:, 2, :]
    iota = lax.broadcasted_iota(jnp.int32, (B, N), 1)
    def _t(col):
        return col.reshape(1, B)

    def step(i, carry):
        distance, farthest = carry
        sel = iota == farthest
        cx = jnp.sum(jnp.where(sel, x, 0.0), axis=1, keepdims=True)
        cy = jnp.sum(jnp.where(sel, y, 0.0), axis=1, keepdims=True)
        cz = jnp.sum(jnp.where(sel, z, 0.0), axis=1, keepdims=True)
        cx_ref[pl.ds(i, 1), :] = _t(cx)
        cy_ref[pl.ds(i, 1), :] = _t(cy)
        cz_ref[pl.ds(i, 1), :] = _t(cz)
        dx = x - cx
        dy = y - cy
        dz = z - cz
        sq1 = dx * dx
        sq2 = dy * dy
        sq3 = dz * dz
        dist = (sq1 + sq2) + sq3
        distance = jnp.minimum(distance, dist)
        m = jnp.max(distance, axis=1, keepdims=True)
        farthest = jnp.min(jnp.where(distance == m, iota, N), axis=1, keepdims=True)
        return distance, farthest

    lax.fori_loop(0, S, step,
                  (jnp.full((B, N), 1e10, jnp.float32),
                   jnp.zeros((B, 1), jnp.int32)))


def _run_fps(xyz):
    return pl.pallas_call(
        _fps_kernel,
        out_shape=[jax.ShapeDtypeStruct((S, B), jnp.float32)] * 3,
    )(xyz)


def _sqr_kernel(nx_ref, xt_ref, ns_ref, nd_ref, o_ref):
    nx = nx_ref[0]
    xt = xt_ref[0]
    mm = lax.dot_general(nx, xt, (((1,), (1,)), ((), ())),
                         preferred_element_type=jnp.float32)
    o_ref[0] = (-2.0 * mm + ns_ref[0, 0][:, None]) + nd_ref[0, 0][None, :]


def _run_sqr(new_xyz, xyz_t, ns_host, nd_host):
    return pl.pallas_call(
        _sqr_kernel,
        grid=(B,),
        in_specs=[
            pl.BlockSpec((1, S, 3), lambda b: (b, 0, 0)),
            pl.BlockSpec((1, N, 3), lambda b: (b, 0, 0)),
            pl.BlockSpec((1, 1, S), lambda b: (b, 0, 0)),
            pl.BlockSpec((1, 1, N), lambda b: (b, 0, 0)),
        ],
        out_specs=pl.BlockSpec((1, S, N), lambda b: (b, 0, 0)),
        out_shape=jax.ShapeDtypeStruct((B, S, N), jnp.float32),
    )(new_xyz, xyz_t, ns_host.reshape(B, 1, S), nd_host.reshape(B, 1, N))


NW = 32
RPW = (B * S) // NW
BLK = 1


DUMP = RPW * K


def _select_body(sqr_hbm, out_hbm, rows_v, outb_v, tmp_v):
    c = lax.axis_index("c")
    s = lax.axis_index("s")
    wid = s * 2 + c
    base_row = wid * RPW

    def row_block(rb, carry):
        row0 = base_row + rb * BLK
        pltpu.sync_copy(sqr_hbm.at[pl.ds(row0 * N, BLK * N)], rows_v)
        b_off = (row0 // S) * N

        for r8 in range(BLK):
            r_loc = rb * BLK + r8
            obase = r_loc * K

            def chunk(ci, cc_vec, _r8=r8, _ob=obase, _bo=b_off):
                iota16 = lax.iota(jnp.int32, 16)
                one = jnp.full((16,), 1.0, jnp.float32)
                fifteen = jnp.full((16,), 15, jnp.int32)
                for u in range(8):
                    off = ci * 128 + u * 16
                    d = rows_v[pl.ds(_r8 * N + off, 16)]
                    indf = jnp.minimum(jnp.maximum((one - d) * 1e30 + one, 0.0), 1.0)
                    mi = indf.astype(jnp.int32)
                    cs = plsc.cumsum(mi)
                    slot = cc_vec + cs - 1
                    sf = (jnp.full((16,), K, jnp.float32) - 0.5) - slot.astype(jnp.float32)
                    okf = jnp.minimum(jnp.maximum(sf * 1e30, 0.0), 1.0)
                    take = mi * okf.astype(jnp.int32)
                    slot_abs = ((jnp.full((16,), _ob, jnp.int32) + slot) * take
                                + (1 - take) * (DUMP + iota16))
                    val = jnp.full((16,), off + _bo, jnp.int32) + iota16
                    plsc.store_scatter(outb_v, [slot_abs], val)
                    tmp_v[pl.ds(0, 16)] = cs
                    cc_vec = cc_vec + tmp_v[pl.ds(0, 16)]
                return cc_vec

            cnt_vec = lax.fori_loop(0, N // 128, chunk, jnp.zeros((16,), jnp.int32))
            tot_splat = jnp.minimum(cnt_vec, K)
            first = outb_v[pl.ds(0, 16)]
            iota16b = lax.iota(jnp.int32, 16)
            for h in range(K // 16):
                pos = iota16b + (h * 16)
                wf = jnp.minimum(jnp.maximum(
                    (tot_splat - pos).astype(jnp.float32) * 1e30, 0.0), 1.0)
                wi = wf.astype(jnp.int32)
                cur = outb_v[pl.ds(obase + h * 16, 16)]
                outb_v[pl.ds(obase + h * 16, 16)] = cur * wi + first * (1 - wi)
        return carry

    lax.fori_loop(0, RPW // BLK, row_block, 0)
    pltpu.sync_copy(outb_v.at[pl.ds(0, RPW * K)], out_hbm.at[pl.ds(base_row * K, RPW * K)])


def _run_select(sqr_flat):
    return pl.kernel(
        _select_body,
        out_type=jax.ShapeDtypeStruct((B * S * K,), jnp.int32),
        mesh=plsc.VectorSubcoreMesh(core_axis_name="c", subcore_axis_name="s"),
        scratch_types=[
            pltpu.VMEM((BLK * N,), jnp.float32),
            pltpu.VMEM((RPW * K + 16,), jnp.int32),
            pltpu.VMEM((16,), jnp.int32),
        ],
    )(sqr_flat)


GRPW = (B * S * K) // NW
GCH = 512


def _gather_body(tab, idx_hbm, out, idx_v, rows_v, sem):
    c = lax.axis_index("c")
    s = lax.axis_index("s")
    wid = s * 2 + c
    wbase = wid * GRPW

    def chunk(ci, carry):
        base = wbase + ci * GCH
        pltpu.sync_copy(idx_hbm.at[pl.ds(base, GCH)], idx_v)
        pltpu.async_copy(tab.at[idx_v], rows_v, sem).wait()
        pltpu.sync_copy(rows_v, out.at[pl.ds(base, GCH)])
        return carry

    lax.fori_loop(0, GRPW // GCH, chunk, 0)


def _run_gather(tab, idx_flat):
    return pl.kernel(
        _gather_body,
        out_type=jax.ShapeDtypeStruct((B * S * K, 128), jnp.float32),
        mesh=plsc.VectorSubcoreMesh(core_axis_name="c", subcore_axis_name="s"),
        scratch_types=[
            pltpu.VMEM((GCH,), jnp.int32),
            pltpu.VMEM((GCH, 128), jnp.float32),
            pltpu.SemaphoreType.DMA,
        ],
    )(tab, idx_flat)


def _index_points3(points, idx):
    return points[jnp.arange(B)[:, None, None], idx]


def _bn(x, g, b):
    mean = jnp.mean(x, axis=(0, 2, 3), keepdims=True)
    var = jnp.var(x, axis=(0, 2, 3), keepdims=True)
    return (x - mean) / jnp.sqrt(var + 1e-5) * g.reshape(1, -1, 1, 1) + b.reshape(1, -1, 1, 1)


def _conv1x1(x, w, b):
    return jnp.einsum('bchw,oc->bohw', x, w[:, :, 0, 0]) + b.reshape(1, -1, 1, 1)


def kernel(xyz, points, npoint, radius, nsample, w1, b1, bn1_g, bn1_b, nt1_w, nt1_b, nt1_g, nt1_b2, nt2_w, nt2_b, nt2_g, nt2_b2, out_w, out_b, out_g, out_b2):
    xyz_t = jnp.transpose(xyz, (0, 2, 1))
    points_t = jnp.transpose(points, (0, 2, 1))

    cx, cy, cz = _run_fps(xyz)
    new_xyz = jnp.stack([cx.T, cy.T, cz.T], axis=-1)

    ns_host = jnp.sum(new_xyz ** 2, -1)
    nd_host = jnp.sum(xyz_t ** 2, -1)
    sqr = _run_sqr(new_xyz, xyz_t, ns_host, nd_host)

    key = jnp.where(sqr > radius ** 2, jnp.float32(-N),
                    -jnp.broadcast_to(jnp.arange(N, dtype=jnp.float32), (B, S, N)))
    vals, _ = lax.top_k(key, K)
    gi = (-vals).astype(jnp.int32)
    first = gi[:, :, :1]
    idx = jnp.where(gi == N, jnp.broadcast_to(first, gi.shape), gi)
    idxg = (idx + (jnp.arange(B, dtype=jnp.int32) * N)[:, None, None]).reshape(B * S, K)

    tab = jnp.concatenate(
        [points_t.reshape(B * N, D), xyz_t.reshape(B * N, 3),
         jnp.zeros((B * N, 128 - D - 3), jnp.float32)], axis=1)
    g_flat = _run_gather(tab, idxg.reshape(B * S * K))
    grouped_points = g_flat[:, :D].reshape(B, S, K, D)
    grouped_xyz = g_flat[:, D:D + 3].reshape(B, S, K, 3)
    gx = grouped_xyz.at[:, :, 0, :].set(0.0)
    density = jnp.sum(gx, axis=-1, keepdims=True)
    density = jnp.where(density < 1e-10, 1e-10, density)
    inv = 1.0 / density
    inv_max = jnp.max(inv, axis=2, keepdims=True)
    density_scale = inv / inv_max
    gxp = jnp.transpose(gx, (0, 3, 1, 2))
    weight = jax.nn.relu(_bn(_conv1x1(gxp, w1, b1), bn1_g, bn1_b))
    ds = jnp.transpose(density_scale, (0, 3, 1, 2))
    ds1 = jax.nn.relu(_bn(_conv1x1(ds, nt1_w, nt1_b), nt1_g, nt1_b2))
    ds = jax.nn.sigmoid(_bn(_conv1x1(ds1, nt2_w, nt2_b), nt2_g, nt2_b2))
    gf = jnp.transpose(grouped_points, (0, 3, 2, 1))
    gf = jnp.transpose(gf, (0, 1, 3, 2))
    npts = gf * ds
    npts = jnp.transpose(npts, (0, 2, 1, 3))
    wgt = jnp.transpose(weight, (0, 2, 3, 1))
    npts = jnp.matmul(npts, wgt)
    npts = jnp.transpose(npts, (0, 2, 1, 3))
    out = jnp.einsum('bcsk,ock->bos', npts, out_w[:, :, 0, :]) + out_b.reshape(1, -1, 1)
    out = out[:, :, :, None]
    out = _bn(out, out_g, out_b2)
    out = jnp.squeeze(out, axis=-1)
    return out


# speedup vs baseline: 2.9622x; 1.0005x over previous
import functools

import jax
import jax.numpy as jnp
from jax import lax
from jax.experimental import pallas as pl
from jax.experimental.pallas import tpu as pltpu
from jax.experimental.pallas import tpu_sc as plsc

B, N, S, K, D = 16, 4096, 512, 32, 64


def _fps_kernel(xyz_ref, cx_ref, cy_ref, cz_ref):
    x = xyz_ref[:, 0, :]
    y = xyz_ref[:, 1, :]
    z = xyz_ref[:, 2, :]
    iota = lax.broadcasted_iota(jnp.int32, (B, N), 1)
    def _t(col):
        return col.reshape(1, B)

    def step(i, carry):
        distance, farthest = carry
        sel = iota == farthest
        cx = jnp.sum(jnp.where(sel, x, 0.0), axis=1, keepdims=True)
        cy = jnp.sum(jnp.where(sel, y, 0.0), axis=1, keepdims=True)
        cz = jnp.sum(jnp.where(sel, z, 0.0), axis=1, keepdims=True)
        cx_ref[pl.ds(i, 1), :] = _t(cx)
        cy_ref[pl.ds(i, 1), :] = _t(cy)
        cz_ref[pl.ds(i, 1), :] = _t(cz)
        dx = x - cx
        dy = y - cy
        dz = z - cz
        sq1 = dx * dx
        sq2 = dy * dy
        sq3 = dz * dz
        dist = (sq1 + sq2) + sq3
        distance = jnp.minimum(distance, dist)
        m = jnp.max(distance, axis=1, keepdims=True)
        farthest = jnp.min(jnp.where(distance == m, iota, N), axis=1, keepdims=True)
        return distance, farthest

    lax.fori_loop(0, S, step,
                  (jnp.full((B, N), 1e10, jnp.float32),
                   jnp.zeros((B, 1), jnp.int32)))


def _run_fps(xyz):
    return pl.pallas_call(
        _fps_kernel,
        out_shape=[jax.ShapeDtypeStruct((S, B), jnp.float32)] * 3,
    )(xyz)


def _sqr_kernel(nx_ref, xt_ref, ns_ref, nd_ref, o_ref):
    nx = nx_ref[0]
    xt = xt_ref[0]
    mm = lax.dot_general(nx, xt, (((1,), (1,)), ((), ())),
                         preferred_element_type=jnp.float32)
    o_ref[0] = (-2.0 * mm + ns_ref[0, 0][:, None]) + nd_ref[0, 0][None, :]


def _run_sqr(new_xyz, xyz_t, ns_host, nd_host):
    return pl.pallas_call(
        _sqr_kernel,
        grid=(B,),
        in_specs=[
            pl.BlockSpec((1, S, 3), lambda b: (b, 0, 0)),
            pl.BlockSpec((1, N, 3), lambda b: (b, 0, 0)),
            pl.BlockSpec((1, 1, S), lambda b: (b, 0, 0)),
            pl.BlockSpec((1, 1, N), lambda b: (b, 0, 0)),
        ],
        out_specs=pl.BlockSpec((1, S, N), lambda b: (b, 0, 0)),
        out_shape=jax.ShapeDtypeStruct((B, S, N), jnp.float32),
    )(new_xyz, xyz_t, ns_host.reshape(B, 1, S), nd_host.reshape(B, 1, N))


NW = 32
RPW = (B * S) // NW
BLK = 1


DUMP = RPW * K


def _select_body(sqr_hbm, out_hbm, rows_v, outb_v, tmp_v):
    c = lax.axis_index("c")
    s = lax.axis_index("s")
    wid = s * 2 + c
    base_row = wid * RPW

    def row_block(rb, carry):
        row0 = base_row + rb * BLK
        pltpu.sync_copy(sqr_hbm.at[pl.ds(row0 * N, BLK * N)], rows_v)
        b_off = (row0 // S) * N

        for r8 in range(BLK):
            r_loc = rb * BLK + r8
            obase = r_loc * K

            def chunk(ci, cc_vec, _r8=r8, _ob=obase, _bo=b_off):
                iota16 = lax.iota(jnp.int32, 16)
                one = jnp.full((16,), 1.0, jnp.float32)
                fifteen = jnp.full((16,), 15, jnp.int32)
                for u in range(8):
                    off = ci * 128 + u * 16
                    d = rows_v[pl.ds(_r8 * N + off, 16)]
                    indf = jnp.minimum(jnp.maximum((one - d) * 1e30 + one, 0.0), 1.0)
                    mi = indf.astype(jnp.int32)
                    cs = plsc.cumsum(mi)
                    slot = cc_vec + cs - 1
                    sf = (jnp.full((16,), K, jnp.float32) - 0.5) - slot.astype(jnp.float32)
                    okf = jnp.minimum(jnp.maximum(sf * 1e30, 0.0), 1.0)
                    take = mi * okf.astype(jnp.int32)
                    slot_abs = ((jnp.full((16,), _ob, jnp.int32) + slot) * take
                                + (1 - take) * (DUMP + iota16))
                    val = jnp.full((16,), off + _bo, jnp.int32) + iota16
                    plsc.store_scatter(outb_v, [slot_abs], val)
                    tmp_v[pl.ds(0, 16)] = cs
                    cc_vec = cc_vec + tmp_v[pl.ds(0, 16)]
                return cc_vec

            cnt_vec = lax.fori_loop(0, N // 128, chunk, jnp.zeros((16,), jnp.int32))
            tot_splat = jnp.minimum(cnt_vec, K)
            first = outb_v[pl.ds(0, 16)]
            iota16b = lax.iota(jnp.int32, 16)
            for h in range(K // 16):
                pos = iota16b + (h * 16)
                wf = jnp.minimum(jnp.maximum(
                    (tot_splat - pos).astype(jnp.float32) * 1e30, 0.0), 1.0)
                wi = wf.astype(jnp.int32)
                cur = outb_v[pl.ds(obase + h * 16, 16)]
                outb_v[pl.ds(obase + h * 16, 16)] = cur * wi + first * (1 - wi)
        return carry

    lax.fori_loop(0, RPW // BLK, row_block, 0)
    pltpu.sync_copy(outb_v.at[pl.ds(0, RPW * K)], out_hbm.at[pl.ds(base_row * K, RPW * K)])


def _run_select(sqr_flat):
    return pl.kernel(
        _select_body,
        out_type=jax.ShapeDtypeStruct((B * S * K,), jnp.int32),
        mesh=plsc.VectorSubcoreMesh(core_axis_name="c", subcore_axis_name="s"),
        scratch_types=[
            pltpu.VMEM((BLK * N,), jnp.float32),
            pltpu.VMEM((RPW * K + 16,), jnp.int32),
            pltpu.VMEM((16,), jnp.int32),
        ],
    )(sqr_flat)


GRPW = (B * S * K) // NW
GCH = 512


def _gather_body(tab, idx_hbm, out, idx_v, rows_v, sem):
    c = lax.axis_index("c")
    s = lax.axis_index("s")
    wid = s * 2 + c
    wbase = wid * GRPW

    def chunk(ci, carry):
        base = wbase + ci * GCH
        pltpu.sync_copy(idx_hbm.at[pl.ds(base, GCH)], idx_v)
        pltpu.async_copy(tab.at[idx_v], rows_v, sem).wait()
        pltpu.sync_copy(rows_v, out.at[pl.ds(base, GCH)])
        return carry

    lax.fori_loop(0, GRPW // GCH, chunk, 0)


def _run_gather(tab, idx_flat):
    return pl.kernel(
        _gather_body,
        out_type=jax.ShapeDtypeStruct((B * S * K, 128), jnp.float32),
        mesh=plsc.VectorSubcoreMesh(core_axis_name="c", subcore_axis_name="s"),
        scratch_types=[
            pltpu.VMEM((GCH,), jnp.int32),
            pltpu.VMEM((GCH, 128), jnp.float32),
            pltpu.SemaphoreType.DMA,
        ],
    )(tab, idx_flat)


def _index_points3(points, idx):
    return points[jnp.arange(B)[:, None, None], idx]


def _bn(x, g, b):
    mean = jnp.mean(x, axis=(0, 2, 3), keepdims=True)
    var = jnp.var(x, axis=(0, 2, 3), keepdims=True)
    return (x - mean) / jnp.sqrt(var + 1e-5) * g.reshape(1, -1, 1, 1) + b.reshape(1, -1, 1, 1)


def _conv1x1(x, w, b):
    return jnp.einsum('bchw,oc->bohw', x, w[:, :, 0, 0]) + b.reshape(1, -1, 1, 1)


def kernel(xyz, points, npoint, radius, nsample, w1, b1, bn1_g, bn1_b, nt1_w, nt1_b, nt1_g, nt1_b2, nt2_w, nt2_b, nt2_g, nt2_b2, out_w, out_b, out_g, out_b2):
    xyz_t = jnp.transpose(xyz, (0, 2, 1))
    points_t = jnp.transpose(points, (0, 2, 1))

    cx, cy, cz = _run_fps(xyz)
    new_xyz = jnp.stack([cx.T, cy.T, cz.T], axis=-1)

    ns_host = jnp.sum(new_xyz ** 2, -1)
    nd_host = jnp.sum(xyz_t ** 2, -1)
    sqr = _run_sqr(new_xyz, xyz_t, ns_host, nd_host)

    key = jnp.where(sqr > radius ** 2, jnp.float32(-N),
                    -jnp.broadcast_to(jnp.arange(N, dtype=jnp.float32), (B, S, N)))
    vals, _ = lax.top_k(key, K)
    gi = (-vals).astype(jnp.int32)
    first = gi[:, :, :1]
    idx = jnp.where(gi == N, jnp.broadcast_to(first, gi.shape), gi)
    idxg = (idx + (jnp.arange(B, dtype=jnp.int32) * N)[:, None, None]).reshape(B * S, K)

    tab = jnp.concatenate(
        [points_t.reshape(B * N, D), xyz_t.reshape(B * N, 3),
         jnp.zeros((B * N, 128 - D - 3), jnp.float32)], axis=1)
    g_flat = _run_gather(tab, idxg.reshape(B * S * K))
    grouped_points = g_flat[:, :D].reshape(B, S, K, D)
    grouped_xyz = g_flat[:, D:D + 3].reshape(B, S, K, 3)
    gx = grouped_xyz.at[:, :, 0, :].set(0.0)
    density = jnp.sum(gx, axis=-1, keepdims=True)
    density = jnp.where(density < 1e-10, 1e-10, density)
    inv = 1.0 / density
    inv_max = jnp.max(inv, axis=2, keepdims=True)
    density_scale = inv / inv_max
    gxp = jnp.transpose(gx, (0, 3, 1, 2))
    weight = jax.nn.relu(_bn(_conv1x1(gxp, w1, b1), bn1_g, bn1_b))
    ds = jnp.transpose(density_scale, (0, 3, 1, 2))
    ds1 = jax.nn.relu(_bn(_conv1x1(ds, nt1_w, nt1_b), nt1_g, nt1_b2))
    ds = jax.nn.sigmoid(_bn(_conv1x1(ds1, nt2_w, nt2_b), nt2_g, nt2_b2))
    dsf = jnp.transpose(ds, (0, 2, 3, 1))
    wgt_t = jnp.transpose(weight, (0, 2, 3, 1))
    p_mat = jnp.einsum('bskd,bskm->bsdm', grouped_points * dsf, wgt_t)
    out = jnp.einsum('bsdm,odm->bos', p_mat, out_w[:, :, 0, :]) + out_b.reshape(1, -1, 1)
    out = out[:, :, :, None]
    out = _bn(out, out_g, out_b2)
    out = jnp.squeeze(out, axis=-1)
    return out
